# baseline (speedup 1.0000x reference)
import functools

import jax
import jax.numpy as jnp
from jax import lax
from jax.experimental import pallas as pl
from jax.experimental.pallas import tpu as pltpu
from jax.experimental.pallas import tpu_sc as plsc

B, C, X, Y = 4, 96, 256, 256
N = 131072
BXY = B * X * Y
NC, NS, L = 2, 16, 16
NW = NC * NS
PTS_PER_W = N // NW
DST_PER_W = BXY // NW
ROWS_PER_W = DST_PER_W // Y
PCHUNK = 8192
K = 8
RPD = Y // K
GPD = RPD // L
CP = 128


def _p_index_body(coords_hbm, p_hbm, cbuf, pout):
    w = lax.axis_index("s") * NC + lax.axis_index("c")
    base = w * PTS_PER_W
    pltpu.sync_copy(coords_hbm.at[pl.ds(base * 3, PTS_PER_W * 3)], cbuf)
    iota = lax.iota(jnp.int32, L)

    @plsc.parallel_loop(0, PTS_PER_W // L, unroll=4)
    def _(j):
        r3 = (j * L + iota) * 3
        bb = plsc.load_gather(cbuf, [r3])
        xx = plsc.load_gather(cbuf, [r3 + 1])
        yy = plsc.load_gather(cbuf, [r3 + 2])
        pout[pl.ds(j * L, L)] = (bb * X + xx) * Y + yy

    pltpu.sync_copy(pout, p_hbm.at[pl.ds(base, PTS_PER_W)])


def _dense_body(p_hbm, feats_hbm, out_hbm, idxbuf, pbuf, rows, outb, fidx,
                maskf, sem):
    w = lax.axis_index("s") * NC + lax.axis_index("c")
    dbase = w * DST_PER_W
    b = w // (X // ROWS_PER_W)
    x0 = (w % (X // ROWS_PER_W)) * ROWS_PER_W
    iota = lax.iota(jnp.int32, L)
    zero16 = jnp.zeros((L,), jnp.int32)

    with jax.named_scope("clear"):
        @plsc.parallel_loop(0, DST_PER_W // L, unroll=8)
        def _(g):
            idxbuf[pl.ds(g * L, L)] = zero16

    with jax.named_scope("scan"):
        for chunk in range(N // PCHUNK):
            pltpu.sync_copy(p_hbm.at[pl.ds(chunk * PCHUNK, PCHUNK)], pbuf)
            cbase = chunk * PCHUNK + 1

            @plsc.parallel_loop(0, PCHUNK // L, unroll=4)
            def _(j):
                v = pbuf[pl.ds(j * L, L)]
                rel = v - dbase
                m = (rel >= 0) & (rel < DST_PER_W)
                relc = jnp.clip(rel, 0, DST_PER_W - 1)
                plsc.store_scatter(idxbuf, [relc], cbase + j * L + iota,
                                   mask=m)

    def prep(ring, sb):
        rbase = sb * Y
        for g in range(Y // L):
            v = idxbuf[pl.ds(rbase + g * L, L)]
            fidx[ring, g // GPD, pl.ds((g % GPD) * L, L)] = \
                jnp.maximum(v - 1, 0)
            maskf[ring, pl.ds(g * L, L)] = jnp.where(v > 0, 1.0, 0.0)

    def fire(ring):
        for q in range(K):
            pltpu.async_copy(feats_hbm.at[fidx.at[ring, q]],
                             rows.at[ring, pl.ds(q * RPD, RPD), :], sem)

    def drain(ring):
        for q in range(K):
            pltpu.make_async_copy(feats_hbm.at[fidx.at[ring, q]],
                                  rows.at[ring, pl.ds(q * RPD, RPD), :],
                                  sem).wait()

    def flush(ring, sb):
        with jax.named_scope("gwait"):
            drain(ring)
        with jax.named_scope("transpose"):
            def g_body(g, _):
                mv = maskf[ring, pl.ds(g * L, L)]
                posv = g * L + iota

                @plsc.parallel_loop(0, C, unroll=8)
                def _(c):
                    vals = plsc.load_gather(rows.at[ring], [posv, zero16 + c])
                    outb[c, pl.ds(g * L, L)] = vals * mv

                return 0

            lax.fori_loop(0, Y // L, g_body, 0)
        with jax.named_scope("writeout"):
            pltpu.sync_copy(outb, out_hbm.at[b, :, x0 + sb, :])

    with jax.named_scope("prime"):
        prep(0, 0)
        fire(0)

    def pair_body(t, _):
        sb0 = 2 * t
        with jax.named_scope("prep"):
            prep(1, sb0 + 1)
        fire(1)
        flush(0, sb0)

        @pl.when(t < ROWS_PER_W // 2 - 1)
        def _():
            with jax.named_scope("prep"):
                prep(0, sb0 + 2)
            fire(0)

        flush(1, sb0 + 1)
        return 0

    lax.fori_loop(0, ROWS_PER_W // 2, pair_body, 0)


@functools.cache
def _build():
    mesh = plsc.VectorSubcoreMesh(core_axis_name="c", subcore_axis_name="s")
    cparams = pltpu.CompilerParams(needs_layout_passes=False,
                                   use_tc_tiling_on_sc=False)
    k1 = pl.kernel(
        _p_index_body,
        out_type=jax.ShapeDtypeStruct((N,), jnp.int32),
        mesh=mesh,
        compiler_params=cparams,
        scratch_types=[
            pltpu.VMEM((PTS_PER_W * 3,), jnp.int32),
            pltpu.VMEM((PTS_PER_W,), jnp.int32),
        ],
    )
    k2 = pl.kernel(
        _dense_body,
        out_type=jax.ShapeDtypeStruct((B, C, X, Y), jnp.float32),
        mesh=mesh,
        compiler_params=cparams,
        scratch_types=[
            pltpu.VMEM((DST_PER_W,), jnp.int32),
            pltpu.VMEM((PCHUNK,), jnp.int32),
            pltpu.VMEM((2, Y, CP), jnp.float32),
            pltpu.VMEM((C, Y), jnp.float32),
            pltpu.VMEM((2, K, RPD), jnp.int32),
            pltpu.VMEM((2, Y), jnp.float32),
            pltpu.SemaphoreType.DMA,
        ],
    )
    return k1, k2


def kernel(feats, coords):
    k1, k2 = _build()
    coords_flat = coords.astype(jnp.int32).reshape(-1)
    feats_pad = jnp.pad(feats, ((0, 0), (0, CP - C)))
    p = k1(coords_flat)
    return k2(p, feats_pad)


# speedup vs baseline: 6.1653x; 6.1653x over previous
import functools

import jax
import jax.numpy as jnp
from jax import lax
from jax.experimental import pallas as pl
from jax.experimental.pallas import tpu as pltpu
from jax.experimental.pallas import tpu_sc as plsc

B, C, X, Y = 4, 96, 256, 256
N = 131072
BXY = B * X * Y
NC, NS, L = 2, 16, 16
NW = NC * NS
PTS_PER_W = N // NW
DST_PER_W = BXY // NW
ROWS_PER_W = DST_PER_W // Y
PCHUNK = 8192
SCH = 512


def _p_index_body(coords_hbm, p_hbm, cbuf, pout):
    w = lax.axis_index("s") * NC + lax.axis_index("c")
    base = w * PTS_PER_W
    pltpu.sync_copy(coords_hbm.at[pl.ds(base * 3, PTS_PER_W * 3)], cbuf)
    iota = lax.iota(jnp.int32, L)

    @plsc.parallel_loop(0, PTS_PER_W // L, unroll=4)
    def _(j):
        r3 = (j * L + iota) * 3
        bb = plsc.load_gather(cbuf, [r3])
        xx = plsc.load_gather(cbuf, [r3 + 1])
        yy = plsc.load_gather(cbuf, [r3 + 2])
        pout[pl.ds(j * L, L)] = (bb * X + xx) * Y + yy

    pltpu.sync_copy(pout, p_hbm.at[pl.ds(base, PTS_PER_W)])


def _scatter_body(p_hbm, feats_hbm, nhwc_hbm, rowbuf, pbuf, sem, ssem):
    w = lax.axis_index("s") * NC + lax.axis_index("c")
    base = w * PTS_PER_W
    iota = lax.iota(jnp.int32, L)

    def stage(ring, ch):
        pltpu.async_copy(
            feats_hbm.at[pl.ds(base + ch * SCH, SCH), :],
            rowbuf.at[ring], ssem)

    def stage_wait(ring, ch):
        pltpu.make_async_copy(
            feats_hbm.at[pl.ds(base + ch * SCH, SCH), :],
            rowbuf.at[ring], ssem).wait()

    def scatter_chunk(ring, ch):
        pltpu.sync_copy(p_hbm.at[pl.ds(base + ch * SCH, SCH)], pbuf)
        stage_wait(ring, ch)

        def g_body(g, _):
            pv = pbuf[pl.ds(g * L, L)]
            for l in range(L):
                pj = jnp.sum(jnp.where(iota == l, pv, 0))
                pltpu.async_copy(rowbuf.at[ring, g * L + l],
                                 nhwc_hbm.at[pj], sem)
            return 0

        lax.fori_loop(0, SCH // L, g_body, 0)

        def d_body(j, _):
            pltpu.make_async_copy(rowbuf.at[ring, 0],
                                  nhwc_hbm.at[0], sem).wait()
            return 0

        lax.fori_loop(0, SCH, d_body, 0)

    stage(0, 0)
    for ch in range(PTS_PER_W // SCH):
        if ch + 1 < PTS_PER_W // SCH:
            stage((ch + 1) % 2, ch + 1)
        scatter_chunk(ch % 2, ch)


def _transpose_body(p_hbm, nhwc_hbm, out_hbm, idxbuf, pbuf, rows, outb, sem):
    w = lax.axis_index("s") * NC + lax.axis_index("c")
    dbase = w * DST_PER_W
    b = w // (X // ROWS_PER_W)
    x0 = (w % (X // ROWS_PER_W)) * ROWS_PER_W
    iota = lax.iota(jnp.int32, L)
    zero16 = jnp.zeros((L,), jnp.int32)
    zf16 = jnp.zeros((L,), jnp.float32)

    with jax.named_scope("clear"):
        @plsc.parallel_loop(0, DST_PER_W // L, unroll=8)
        def _(g):
            idxbuf[pl.ds(g * L, L)] = zero16

    with jax.named_scope("scan"):
        for chunk in range(N // PCHUNK):
            pltpu.sync_copy(p_hbm.at[pl.ds(chunk * PCHUNK, PCHUNK)], pbuf)

            @plsc.parallel_loop(0, PCHUNK // L, unroll=4)
            def _(j):
                v = pbuf[pl.ds(j * L, L)]
                rel = v - dbase
                m = (rel >= 0) & (rel < DST_PER_W)
                relc = jnp.clip(rel, 0, DST_PER_W - 1)
                plsc.store_scatter(idxbuf, [relc], iota + 1, mask=m)

    def stage(ring, sb):
        pltpu.async_copy(
            nhwc_hbm.at[pl.ds(dbase + sb * Y, Y), :], rows.at[ring], sem)

    def stage_wait(ring, sb):
        pltpu.make_async_copy(
            nhwc_hbm.at[pl.ds(dbase + sb * Y, Y), :], rows.at[ring],
            sem).wait()

    def flush(ring, sb):
        with jax.named_scope("gwait"):
            stage_wait(ring, sb)
        rbase = sb * Y
        with jax.named_scope("transpose"):
            @plsc.parallel_loop(0, C * (Y // L), unroll=8)
            def _(idx):
                g = idx & (Y // L - 1)
                c = idx >> 4
                posv = g * L + iota
                ibv = idxbuf[pl.ds(rbase + g * L, L)]
                vals = plsc.load_gather(rows.at[ring], [posv, zero16 + c])
                outb[c, pl.ds(g * L, L)] = jnp.where(ibv > 0, vals, zf16)
        with jax.named_scope("writeout"):
            pltpu.sync_copy(outb, out_hbm.at[b, :, x0 + sb, :])

    with jax.named_scope("prime"):
        stage(0, 0)

    def pair_body(t, _):
        sb0 = 2 * t
        stage(1, sb0 + 1)
        flush(0, sb0)

        @pl.when(t < ROWS_PER_W // 2 - 1)
        def _():
            stage(0, sb0 + 2)

        flush(1, sb0 + 1)
        return 0

    lax.fori_loop(0, ROWS_PER_W // 2, pair_body, 0)


@functools.cache
def _build():
    mesh = plsc.VectorSubcoreMesh(core_axis_name="c", subcore_axis_name="s")
    cparams = pltpu.CompilerParams(needs_layout_passes=False,
                                   use_tc_tiling_on_sc=False)
    k1 = pl.kernel(
        _p_index_body,
        out_type=jax.ShapeDtypeStruct((N,), jnp.int32),
        mesh=mesh,
        compiler_params=cparams,
        scratch_types=[
            pltpu.VMEM((PTS_PER_W * 3,), jnp.int32),
            pltpu.VMEM((PTS_PER_W,), jnp.int32),
        ],
    )
    k2 = pl.kernel(
        _scatter_body,
        out_type=jax.ShapeDtypeStruct((BXY, C), jnp.float32),
        mesh=mesh,
        compiler_params=cparams,
        scratch_types=[
            pltpu.VMEM((2, SCH, C), jnp.float32),
            pltpu.VMEM((SCH,), jnp.int32),
            pltpu.SemaphoreType.DMA,
            pltpu.SemaphoreType.DMA,
        ],
    )
    k3 = pl.kernel(
        _transpose_body,
        out_type=jax.ShapeDtypeStruct((B, C, X, Y), jnp.float32),
        mesh=mesh,
        compiler_params=cparams,
        scratch_types=[
            pltpu.VMEM((DST_PER_W,), jnp.int32),
            pltpu.VMEM((PCHUNK,), jnp.int32),
            pltpu.VMEM((2, Y, C), jnp.float32),
            pltpu.VMEM((C, Y), jnp.float32),
            pltpu.SemaphoreType.DMA,
        ],
    )
    return k1, k2, k3


def kernel(feats, coords):
    k1, k2, k3 = _build()
    coords_flat = coords.astype(jnp.int32).reshape(-1)
    p = k1(coords_flat)
    nhwc = k2(p, feats)
    return k3(p, nhwc)


# speedup vs baseline: 9.9296x; 1.6106x over previous
import functools

import jax
import jax.numpy as jnp
from jax import lax
from jax.experimental import pallas as pl
from jax.experimental.pallas import tpu as pltpu
from jax.experimental.pallas import tpu_sc as plsc

B, C, X, Y = 4, 96, 256, 256
N = 131072
BXY = B * X * Y
NC, NS, L = 2, 16, 16
NW = NC * NS
PTS_PER_W = N // NW
DST_PER_W = BXY // NW
ROWS_PER_W = DST_PER_W // Y
PCHUNK = 8192
SCH = 512


def _p_index_body(coords_hbm, p_hbm, cbuf, pout):
    w = lax.axis_index("s") * NC + lax.axis_index("c")
    base = w * PTS_PER_W
    pltpu.sync_copy(coords_hbm.at[pl.ds(base * 3, PTS_PER_W * 3)], cbuf)
    iota = lax.iota(jnp.int32, L)

    @plsc.parallel_loop(0, PTS_PER_W // L, unroll=4)
    def _(j):
        r3 = (j * L + iota) * 3
        bb = plsc.load_gather(cbuf, [r3])
        xx = plsc.load_gather(cbuf, [r3 + 1])
        yy = plsc.load_gather(cbuf, [r3 + 2])
        pout[pl.ds(j * L, L)] = (bb * X + xx) * Y + yy

    pltpu.sync_copy(pout, p_hbm.at[pl.ds(base, PTS_PER_W)])


def _scatter_body(p_hbm, feats_hbm, nhwc_hbm, rowbuf, pbuf, sem, ssem):
    w = lax.axis_index("s") * NC + lax.axis_index("c")
    base = w * PTS_PER_W
    iota = lax.iota(jnp.int32, L)

    def stage(ring, ch):
        pltpu.async_copy(
            feats_hbm.at[pl.ds(base + ch * SCH, SCH), :],
            rowbuf.at[ring], ssem)

    def stage_wait(ring, ch):
        pltpu.make_async_copy(
            feats_hbm.at[pl.ds(base + ch * SCH, SCH), :],
            rowbuf.at[ring], ssem).wait()

    def scatter_chunk(ring, ch):
        pltpu.sync_copy(p_hbm.at[pl.ds(base + ch * SCH, SCH)], pbuf)
        stage_wait(ring, ch)

        def g_body(g, _):
            pv = pbuf[pl.ds(g * L, L)]
            for l in range(L):
                pj = jnp.sum(jnp.where(iota == l, pv, 0))
                pltpu.async_copy(rowbuf.at[ring, g * L + l],
                                 nhwc_hbm.at[pj], sem)
            return 0

        lax.fori_loop(0, SCH // L, g_body, 0)

        def d_body(j, _):
            pltpu.make_async_copy(rowbuf.at[ring, 0],
                                  nhwc_hbm.at[0], sem).wait()
            return 0

        lax.fori_loop(0, SCH, d_body, 0)

    stage(0, 0)
    for ch in range(PTS_PER_W // SCH):
        if ch + 1 < PTS_PER_W // SCH:
            stage((ch + 1) % 2, ch + 1)
        scatter_chunk(ch % 2, ch)


def _transpose_body(p_hbm, nhwc_hbm, out_hbm, idxbuf, pbuf, rows, outb, sem):
    w = lax.axis_index("s") * NC + lax.axis_index("c")
    dbase = w * DST_PER_W
    b = w // (X // ROWS_PER_W)
    x0 = (w % (X // ROWS_PER_W)) * ROWS_PER_W
    iota = lax.iota(jnp.int32, L)
    zero16 = jnp.zeros((L,), jnp.int32)
    zf16 = jnp.zeros((L,), jnp.float32)

    with jax.named_scope("clear"):
        @plsc.parallel_loop(0, DST_PER_W // L, unroll=8)
        def _(g):
            idxbuf[pl.ds(g * L, L)] = zero16

    with jax.named_scope("scan"):
        for chunk in range(N // PCHUNK):
            pltpu.sync_copy(p_hbm.at[pl.ds(chunk * PCHUNK, PCHUNK)], pbuf)

            @plsc.parallel_loop(0, PCHUNK // L, unroll=4)
            def _(j):
                v = pbuf[pl.ds(j * L, L)]
                rel = v - dbase
                m = (rel >= 0) & (rel < DST_PER_W)
                relc = jnp.clip(rel, 0, DST_PER_W - 1)
                plsc.store_scatter(idxbuf, [relc], iota + 1, mask=m)

    def stage(ring, sb):
        pltpu.async_copy(
            nhwc_hbm.at[pl.ds(dbase + sb * Y, Y), :], rows.at[ring], sem)

    def stage_wait(ring, sb):
        pltpu.make_async_copy(
            nhwc_hbm.at[pl.ds(dbase + sb * Y, Y), :], rows.at[ring],
            sem).wait()

    def flush(ring, sb):
        with jax.named_scope("gwait"):
            stage_wait(ring, sb)
        rbase = sb * Y
        with jax.named_scope("transpose"):
            for cg in range(C // L):
                c0 = cg * L

                @plsc.parallel_loop(0, Y // L, unroll=2)
                def _(g):
                    posv = g * L + iota
                    ibv = idxbuf[pl.ds(rbase + g * L, L)]
                    m = ibv > 0
                    for d in range(L):
                        ch = (iota + d) & (L - 1)
                        vals = plsc.load_gather(rows.at[ring],
                                                [posv, c0 + ch])
                        plsc.store_scatter(outb, [c0 + ch, posv],
                                           jnp.where(m, vals, zf16))
        with jax.named_scope("writeout"):
            pltpu.sync_copy(outb, out_hbm.at[b, :, x0 + sb, :])

    with jax.named_scope("prime"):
        stage(0, 0)

    def pair_body(t, _):
        sb0 = 2 * t
        stage(1, sb0 + 1)
        flush(0, sb0)

        @pl.when(t < ROWS_PER_W // 2 - 1)
        def _():
            stage(0, sb0 + 2)

        flush(1, sb0 + 1)
        return 0

    lax.fori_loop(0, ROWS_PER_W // 2, pair_body, 0)


@functools.cache
def _build():
    mesh = plsc.VectorSubcoreMesh(core_axis_name="c", subcore_axis_name="s")
    cparams = pltpu.CompilerParams(needs_layout_passes=False,
                                   use_tc_tiling_on_sc=False)
    k1 = pl.kernel(
        _p_index_body,
        out_type=jax.ShapeDtypeStruct((N,), jnp.int32),
        mesh=mesh,
        compiler_params=cparams,
        scratch_types=[
            pltpu.VMEM((PTS_PER_W * 3,), jnp.int32),
            pltpu.VMEM((PTS_PER_W,), jnp.int32),
        ],
    )
    k2 = pl.kernel(
        _scatter_body,
        out_type=jax.ShapeDtypeStruct((BXY, C), jnp.float32),
        mesh=mesh,
        compiler_params=cparams,
        scratch_types=[
            pltpu.VMEM((2, SCH, C), jnp.float32),
            pltpu.VMEM((SCH,), jnp.int32),
            pltpu.SemaphoreType.DMA,
            pltpu.SemaphoreType.DMA,
        ],
    )
    k3 = pl.kernel(
        _transpose_body,
        out_type=jax.ShapeDtypeStruct((B, C, X, Y), jnp.float32),
        mesh=mesh,
        compiler_params=cparams,
        scratch_types=[
            pltpu.VMEM((DST_PER_W,), jnp.int32),
            pltpu.VMEM((PCHUNK,), jnp.int32),
            pltpu.VMEM((2, Y, C), jnp.float32),
            pltpu.VMEM((C, Y), jnp.float32),
            pltpu.SemaphoreType.DMA,
        ],
    )
    return k1, k2, k3


def kernel(feats, coords):
    k1, k2, k3 = _build()
    coords_flat = coords.astype(jnp.int32).reshape(-1)
    p = k1(coords_flat)
    nhwc = k2(p, feats)
    return k3(p, nhwc)


# speedup vs baseline: 11.5655x; 1.1648x over previous
import functools

import jax
import jax.numpy as jnp
from jax import lax
from jax.experimental import pallas as pl
from jax.experimental.pallas import tpu as pltpu
from jax.experimental.pallas import tpu_sc as plsc

B, C, X, Y = 4, 96, 256, 256
N = 131072
BXY = B * X * Y
NC, NS, L = 2, 16, 16
NW = NC * NS
PTS_PER_W = N // NW
DST_PER_W = BXY // NW
ROWS_PER_W = DST_PER_W // Y
PCHUNK = 8192
SCH = 512
CCH = 1024


def _scatter_body(coords_hbm, feats_hbm, nhwc_hbm, p_hbm, cbuf, rowbuf,
                  pbuf, sem, ssem):
    w = lax.axis_index("s") * NC + lax.axis_index("c")
    base = w * PTS_PER_W
    iota = lax.iota(jnp.int32, L)
    zero16 = jnp.zeros((L,), jnp.int32)

    def stage(ring, ch):
        pltpu.async_copy(
            feats_hbm.at[pl.ds(base + ch * SCH, SCH), :],
            rowbuf.at[ring], ssem)

    def stage_wait(ring, ch):
        pltpu.make_async_copy(
            feats_hbm.at[pl.ds(base + ch * SCH, SCH), :],
            rowbuf.at[ring], ssem).wait()

    stage(0, 0)
    for cc in range(PTS_PER_W // CCH):
        pltpu.sync_copy(coords_hbm.at[pl.ds(base + cc * CCH, CCH), :], cbuf)

        @plsc.parallel_loop(0, CCH // L, unroll=4)
        def _(j):
            rows16 = j * L + iota
            bb = plsc.load_gather(cbuf, [rows16, zero16])
            xx = plsc.load_gather(cbuf, [rows16, zero16 + 1])
            yy = plsc.load_gather(cbuf, [rows16, zero16 + 2])
            pbuf[pl.ds(cc * CCH + j * L, L)] = (bb * X + xx) * Y + yy

    pltpu.sync_copy(pbuf, p_hbm.at[pl.ds(base, PTS_PER_W)])

    def scatter_chunk(ring, ch):
        stage_wait(ring, ch)
        cb = ch * SCH

        def g_body(g, _):
            pv = pbuf[pl.ds(cb + g * L, L)]
            for l in range(L):
                pj = jnp.sum(jnp.where(iota == l, pv, 0))
                pltpu.async_copy(rowbuf.at[ring, g * L + l],
                                 nhwc_hbm.at[pj], sem)
            return 0

        lax.fori_loop(0, SCH // L, g_body, 0)

        def d_body(j, _):
            pltpu.make_async_copy(rowbuf.at[ring, 0],
                                  nhwc_hbm.at[0], sem).wait()
            return 0

        lax.fori_loop(0, SCH, d_body, 0)

    for ch in range(PTS_PER_W // SCH):
        if ch + 1 < PTS_PER_W // SCH:
            stage((ch + 1) % 2, ch + 1)
        scatter_chunk(ch % 2, ch)


def _transpose_body(p_hbm, nhwc_hbm, out_hbm, idxbuf, pbuf, rows, outb, sem):
    w = lax.axis_index("s") * NC + lax.axis_index("c")
    dbase = w * DST_PER_W
    b = w // (X // ROWS_PER_W)
    x0 = (w % (X // ROWS_PER_W)) * ROWS_PER_W
    iota = lax.iota(jnp.int32, L)
    zero16 = jnp.zeros((L,), jnp.int32)
    zf16 = jnp.zeros((L,), jnp.float32)

    with jax.named_scope("clear"):
        @plsc.parallel_loop(0, DST_PER_W // L, unroll=8)
        def _(g):
            idxbuf[pl.ds(g * L, L)] = zero16

    with jax.named_scope("scan"):
        for chunk in range(N // PCHUNK):
            pltpu.sync_copy(p_hbm.at[pl.ds(chunk * PCHUNK, PCHUNK)], pbuf)

            @plsc.parallel_loop(0, PCHUNK // L, unroll=4)
            def _(j):
                v = pbuf[pl.ds(j * L, L)]
                rel = v - dbase
                m = (rel >= 0) & (rel < DST_PER_W)
                relc = jnp.clip(rel, 0, DST_PER_W - 1)
                plsc.store_scatter(idxbuf, [relc], iota + 1, mask=m)

    def stage(ring, sb):
        pltpu.async_copy(
            nhwc_hbm.at[pl.ds(dbase + sb * Y, Y), :], rows.at[ring], sem)

    def stage_wait(ring, sb):
        pltpu.make_async_copy(
            nhwc_hbm.at[pl.ds(dbase + sb * Y, Y), :], rows.at[ring],
            sem).wait()

    def flush(ring, sb):
        with jax.named_scope("gwait"):
            stage_wait(ring, sb)
        rbase = sb * Y
        with jax.named_scope("transpose"):
            for cg in range(C // L):
                c0 = cg * L

                @plsc.parallel_loop(0, Y // L, unroll=2)
                def _(g):
                    posv = g * L + iota
                    ibv = idxbuf[pl.ds(rbase + g * L, L)]
                    m = ibv > 0
                    for d in range(L):
                        ch = (iota + d) & (L - 1)
                        vals = plsc.load_gather(rows.at[ring],
                                                [posv, c0 + ch])
                        plsc.store_scatter(
                            outb, [c0 + ch, posv >> 7, posv & (128 - 1)],
                            jnp.where(m, vals, zf16))
        with jax.named_scope("writeout"):
            x = x0 + sb
            pltpu.sync_copy(outb, out_hbm.at[b, :, x >> 3, :, x & 7, :])

    with jax.named_scope("prime"):
        stage(0, 0)

    def pair_body(t, _):
        sb0 = 2 * t
        stage(1, sb0 + 1)
        flush(0, sb0)

        @pl.when(t < ROWS_PER_W // 2 - 1)
        def _():
            stage(0, sb0 + 2)

        flush(1, sb0 + 1)
        return 0

    lax.fori_loop(0, ROWS_PER_W // 2, pair_body, 0)


@functools.cache
def _build():
    mesh = plsc.VectorSubcoreMesh(core_axis_name="c", subcore_axis_name="s")
    cparams = pltpu.CompilerParams(needs_layout_passes=False,
                                   use_tc_tiling_on_sc=False)
    k2 = pl.kernel(
        _scatter_body,
        out_type=(
            jax.ShapeDtypeStruct((BXY, C), jnp.float32),
            jax.ShapeDtypeStruct((N,), jnp.int32),
        ),
        mesh=mesh,
        compiler_params=cparams,
        scratch_types=[
            pltpu.VMEM((CCH, 3), jnp.int32),
            pltpu.VMEM((2, SCH, C), jnp.float32),
            pltpu.VMEM((PTS_PER_W,), jnp.int32),
            pltpu.SemaphoreType.DMA,
            pltpu.SemaphoreType.DMA,
        ],
    )
    k3 = pl.kernel(
        _transpose_body,
        out_type=jax.ShapeDtypeStruct((B, C, X // 8, Y // 128, 8, 128),
                                      jnp.float32),
        mesh=mesh,
        compiler_params=cparams,
        scratch_types=[
            pltpu.VMEM((DST_PER_W,), jnp.int32),
            pltpu.VMEM((PCHUNK,), jnp.int32),
            pltpu.VMEM((2, Y, C), jnp.float32),
            pltpu.VMEM((C, Y // 128, 128), jnp.float32),
            pltpu.SemaphoreType.DMA,
        ],
    )
    return k2, k3


def kernel(feats, coords):
    k2, k3 = _build()
    nhwc, p = k2(coords.astype(jnp.int32), feats)
    out6 = k3(p, nhwc)
    return out6.transpose(0, 1, 2, 4, 3, 5).reshape(B, C, X, Y)


# speedup vs baseline: 15.0600x; 1.3021x over previous
import functools

import jax
import jax.numpy as jnp
from jax import lax
from jax.experimental import pallas as pl
from jax.experimental.pallas import tpu as pltpu
from jax.experimental.pallas import tpu_sc as plsc

B, C, X, Y = 4, 96, 256, 256
N = 131072
BXY = B * X * Y
NC, NS, L = 2, 16, 16
NW = NC * NS
PTS_PER_W = N // NW
DST_PER_W = BXY // NW
ROWS_PER_W = DST_PER_W // Y
PCHUNK = 8192
SCH = 512
CCH = 1024


def _scatter_body(coords_hbm, feats_hbm, nhwc_hbm, p_hbm, cbuf, rowbuf,
                  pbuf, sem, ssem):
    w = lax.axis_index("s") * NC + lax.axis_index("c")
    base = w * PTS_PER_W
    iota = lax.iota(jnp.int32, L)
    zero16 = jnp.zeros((L,), jnp.int32)

    def stage(ring, ch):
        pltpu.async_copy(
            feats_hbm.at[pl.ds(base + ch * SCH, SCH), :],
            rowbuf.at[ring], ssem)

    def stage_wait(ring, ch):
        pltpu.make_async_copy(
            feats_hbm.at[pl.ds(base + ch * SCH, SCH), :],
            rowbuf.at[ring], ssem).wait()

    stage(0, 0)
    pltpu.sync_copy(coords_hbm.at[:, pl.ds(base, PTS_PER_W)], cbuf)

    @plsc.parallel_loop(0, PTS_PER_W // L, unroll=4)
    def _(j):
        bb = cbuf[0, pl.ds(j * L, L)]
        xx = cbuf[1, pl.ds(j * L, L)]
        yy = cbuf[2, pl.ds(j * L, L)]
        pbuf[pl.ds(j * L, L)] = (bb * X + xx) * Y + yy

    pltpu.sync_copy(pbuf, p_hbm.at[pl.ds(base, PTS_PER_W)])

    def scatter_chunk(ring, ch):
        stage_wait(ring, ch)
        cb = ch * SCH

        def g_body(g, _):
            pv = pbuf[pl.ds(cb + g * L, L)]
            for l in range(L):
                pj = jnp.sum(jnp.where(iota == l, pv, 0))
                pltpu.async_copy(rowbuf.at[ring, g * L + l],
                                 nhwc_hbm.at[pj], sem)
            return 0

        lax.fori_loop(0, SCH // L, g_body, 0)

        def d_body(j, _):
            pltpu.make_async_copy(rowbuf.at[ring, 0],
                                  nhwc_hbm.at[0], sem).wait()
            return 0

        lax.fori_loop(0, SCH, d_body, 0)

    for ch in range(PTS_PER_W // SCH):
        if ch + 1 < PTS_PER_W // SCH:
            stage((ch + 1) % 2, ch + 1)
        scatter_chunk(ch % 2, ch)


def _transpose_body(p_hbm, nhwc_hbm, out_hbm, idxbuf, pbuf, rows, outb, sem):
    w = lax.axis_index("s") * NC + lax.axis_index("c")
    dbase = w * DST_PER_W
    b = w // (X // ROWS_PER_W)
    x0 = (w % (X // ROWS_PER_W)) * ROWS_PER_W
    iota = lax.iota(jnp.int32, L)
    zero16 = jnp.zeros((L,), jnp.int32)
    zf16 = jnp.zeros((L,), jnp.float32)

    with jax.named_scope("clear"):
        @plsc.parallel_loop(0, DST_PER_W // L, unroll=8)
        def _(g):
            idxbuf[pl.ds(g * L, L)] = zero16

    with jax.named_scope("scan"):
        for chunk in range(N // PCHUNK):
            pltpu.sync_copy(p_hbm.at[pl.ds(chunk * PCHUNK, PCHUNK)], pbuf)

            @plsc.parallel_loop(0, PCHUNK // L, unroll=4)
            def _(j):
                v = pbuf[pl.ds(j * L, L)]
                rel = v - dbase
                m = (rel >= 0) & (rel < DST_PER_W)
                relc = jnp.clip(rel, 0, DST_PER_W - 1)
                plsc.store_scatter(idxbuf, [relc], iota + 1, mask=m)

    def stage(ring, sb):
        pltpu.async_copy(
            nhwc_hbm.at[pl.ds(dbase + sb * Y, Y), :], rows.at[ring], sem)

    def stage_wait(ring, sb):
        pltpu.make_async_copy(
            nhwc_hbm.at[pl.ds(dbase + sb * Y, Y), :], rows.at[ring],
            sem).wait()

    def flush(ring, sb):
        with jax.named_scope("gwait"):
            stage_wait(ring, sb)
        rbase = sb * Y
        with jax.named_scope("transpose"):
            for cg in range(C // L):
                c0 = cg * L

                @plsc.parallel_loop(0, Y // L, unroll=2)
                def _(g):
                    posv = g * L + iota
                    ibv = idxbuf[pl.ds(rbase + g * L, L)]
                    m = ibv > 0
                    for d in range(L):
                        ch = (iota + d) & (L - 1)
                        vals = plsc.load_gather(rows.at[ring],
                                                [posv, c0 + ch])
                        plsc.store_scatter(
                            outb, [c0 + ch, posv >> 7, posv & (128 - 1)],
                            jnp.where(m, vals, zf16))
        with jax.named_scope("writeout"):
            x = x0 + sb
            pltpu.sync_copy(outb, out_hbm.at[b, :, x >> 3, :, x & 7, :])

    with jax.named_scope("prime"):
        stage(0, 0)

    def pair_body(t, _):
        sb0 = 2 * t
        stage(1, sb0 + 1)
        flush(0, sb0)

        @pl.when(t < ROWS_PER_W // 2 - 1)
        def _():
            stage(0, sb0 + 2)

        flush(1, sb0 + 1)
        return 0

    lax.fori_loop(0, ROWS_PER_W // 2, pair_body, 0)


@functools.cache
def _build():
    mesh = plsc.VectorSubcoreMesh(core_axis_name="c", subcore_axis_name="s")
    cparams = pltpu.CompilerParams(needs_layout_passes=False,
                                   use_tc_tiling_on_sc=False)
    k2 = pl.kernel(
        _scatter_body,
        out_type=(
            jax.ShapeDtypeStruct((BXY, C), jnp.float32),
            jax.ShapeDtypeStruct((N,), jnp.int32),
        ),
        mesh=mesh,
        compiler_params=cparams,
        scratch_types=[
            pltpu.VMEM((3, PTS_PER_W), jnp.int32),
            pltpu.VMEM((2, SCH, C), jnp.float32),
            pltpu.VMEM((PTS_PER_W,), jnp.int32),
            pltpu.SemaphoreType.DMA,
            pltpu.SemaphoreType.DMA,
        ],
    )
    k3 = pl.kernel(
        _transpose_body,
        out_type=jax.ShapeDtypeStruct((B, C, X // 8, Y // 128, 8, 128),
                                      jnp.float32),
        mesh=mesh,
        compiler_params=cparams,
        scratch_types=[
            pltpu.VMEM((DST_PER_W,), jnp.int32),
            pltpu.VMEM((PCHUNK,), jnp.int32),
            pltpu.VMEM((2, Y, C), jnp.float32),
            pltpu.VMEM((C, Y // 128, 128), jnp.float32),
            pltpu.SemaphoreType.DMA,
        ],
    )
    return k2, k3


def kernel(feats, coords):
    k2, k3 = _build()
    nhwc, p = k2(coords.astype(jnp.int32).T, feats)
    out6 = k3(p, nhwc)
    return out6.transpose(0, 1, 2, 4, 3, 5).reshape(B, C, X, Y)
